# baseline (speedup 1.0000x reference)
import functools

import jax
import jax.numpy as jnp
from jax import lax
from jax.experimental import pallas as pl
from jax.experimental.pallas import tpu as pltpu
from jax.experimental.pallas import tpu_sc as plsc

BATCH = 4096
SEQ = 50
EMBED_DIM = 64
PAD_DIM = 128
N_ROWS = 1000000

NC = 2
NS = 16
NW = NC * NS
B = BATCH * SEQ
B_PER_W = B // NW
CHUNK = 128
NCHUNK = B_PER_W // CHUNK
NBUF = 5
NOUTER = NCHUNK // NBUF

TCOLS = 2048
NTBLK = -(-N_ROWS // TCOLS)
PAD_ROWS = NTBLK * TCOLS


def _transpose_body(tt_ref, out_ref):
    x = tt_ref[...]
    row = jax.lax.broadcasted_iota(jnp.int32, (EMBED_DIM, EMBED_DIM), 0)
    col = jax.lax.broadcasted_iota(jnp.int32, (EMBED_DIM, EMBED_DIM), 1)
    ident = jnp.where(row == col, 1.0, 0.0).astype(jnp.float32)
    y = jax.lax.dot_general(
        x, ident, (((0,), (0,)), ((), ())),
        preferred_element_type=jnp.float32)
    out_ref[...] = jnp.concatenate(
        [y, jnp.zeros((TCOLS, PAD_DIM - EMBED_DIM), jnp.float32)], axis=1)


@functools.cache
def _make_tc_transpose():
    return pl.pallas_call(
        _transpose_body,
        grid=(NTBLK,),
        in_specs=[pl.BlockSpec((EMBED_DIM, TCOLS), lambda i: (0, i))],
        out_specs=pl.BlockSpec((TCOLS, PAD_DIM), lambda i: (i, 0)),
        out_shape=jax.ShapeDtypeStruct((PAD_ROWS, PAD_DIM), jnp.float32),
    )


def _gather_body(table_hbm, idx_hbm, out_hbm, idx_v, rows_v, gsem, ssem):
    wid = lax.axis_index("s") * NC + lax.axis_index("c")
    base = wid * B_PER_W
    pltpu.sync_copy(idx_hbm.at[wid], idx_v)

    def outer(g, carry):
        for b in range(NBUF):
            @pl.when(g > 0)
            def _wait_store():
                pltpu.make_async_copy(
                    rows_v.at[b], out_hbm.at[pl.ds(base, CHUNK)],
                    ssem.at[b]).wait()
            pltpu.make_async_copy(
                table_hbm.at[idx_v.at[g * NBUF + b]], rows_v.at[b],
                gsem.at[b]).start()
        for b in range(NBUF):
            j = g * NBUF + b
            pltpu.make_async_copy(
                table_hbm.at[idx_v.at[j]], rows_v.at[b], gsem.at[b]).wait()
            pltpu.make_async_copy(
                rows_v.at[b], out_hbm.at[pl.ds(base + j * CHUNK, CHUNK)],
                ssem.at[b]).start()
        return carry

    lax.fori_loop(0, NOUTER, outer, 0)
    for b in range(NBUF):
        pltpu.make_async_copy(
            rows_v.at[b], out_hbm.at[pl.ds(base, CHUNK)], ssem.at[b]).wait()


@functools.cache
def _make_sc_gather():
    return functools.partial(
        pl.kernel,
        mesh=plsc.VectorSubcoreMesh(
            core_axis_name="c", subcore_axis_name="s",
            num_cores=NC, num_subcores=NS),
        out_type=jax.ShapeDtypeStruct((B, PAD_DIM), jnp.float32),
        scratch_types=[
            pltpu.VMEM((NCHUNK, CHUNK), jnp.int32),
            pltpu.VMEM((NBUF, CHUNK, PAD_DIM), jnp.float32),
            pltpu.SemaphoreType.DMA((NBUF,)),
            pltpu.SemaphoreType.DMA((NBUF,)),
        ],
        compiler_params=pltpu.CompilerParams(use_tc_tiling_on_sc=True),
    )(_gather_body)


def kernel(input_ids, table):
    ids = input_ids.astype(jnp.int32).reshape(NW, NCHUNK, CHUNK)
    tpad = _make_tc_transpose()(table.T)
    out = _make_sc_gather()(tpad, ids)
    return out[:, :EMBED_DIM].reshape(BATCH, SEQ, EMBED_DIM)


# speedup vs baseline: 1.4266x; 1.4266x over previous
import functools

import jax
import jax.numpy as jnp
from jax import lax
from jax.experimental import pallas as pl
from jax.experimental.pallas import tpu as pltpu
from jax.experimental.pallas import tpu_sc as plsc

BATCH = 4096
SEQ = 50
EMBED_DIM = 64
PAD_DIM = 128
N_ROWS = 1000000

NC = 2
NS = 16
NW = NC * NS
B = BATCH * SEQ
B_PER_W = B // NW
CHUNK = 128
NCHUNK = B_PER_W // CHUNK
NBUF = 5
NOUTER = NCHUNK // NBUF

TCOLS = 8192
NTBLK = -(-N_ROWS // TCOLS)
PAD_ROWS = NTBLK * TCOLS


def _transpose_body(tt_ref, out_ref):
    x = tt_ref[...]
    out_ref[:, 0:EMBED_DIM] = x.T


@functools.cache
def _make_tc_transpose():
    return pl.pallas_call(
        _transpose_body,
        grid=(NTBLK,),
        in_specs=[pl.BlockSpec((EMBED_DIM, TCOLS), lambda i: (0, i))],
        out_specs=pl.BlockSpec((TCOLS, PAD_DIM), lambda i: (i, 0)),
        out_shape=jax.ShapeDtypeStruct((PAD_ROWS, PAD_DIM), jnp.float32),
    )


def _gather_body(table_hbm, idx_hbm, out_hbm, idx_v, rows_v, gsem, ssem):
    wid = lax.axis_index("s") * NC + lax.axis_index("c")
    base = wid * B_PER_W
    pltpu.sync_copy(idx_hbm.at[wid], idx_v)

    def outer(g, carry):
        for b in range(NBUF):
            @pl.when(g > 0)
            def _wait_store():
                pltpu.make_async_copy(
                    rows_v.at[b], out_hbm.at[pl.ds(base, CHUNK)],
                    ssem.at[b]).wait()
            pltpu.make_async_copy(
                table_hbm.at[idx_v.at[g * NBUF + b]], rows_v.at[b],
                gsem.at[b]).start()
        for b in range(NBUF):
            j = g * NBUF + b
            pltpu.make_async_copy(
                table_hbm.at[idx_v.at[j]], rows_v.at[b], gsem.at[b]).wait()
            pltpu.make_async_copy(
                rows_v.at[b], out_hbm.at[pl.ds(base + j * CHUNK, CHUNK)],
                ssem.at[b]).start()
        return carry

    lax.fori_loop(0, NOUTER, outer, 0)
    for b in range(NBUF):
        pltpu.make_async_copy(
            rows_v.at[b], out_hbm.at[pl.ds(base, CHUNK)], ssem.at[b]).wait()


@functools.cache
def _make_sc_gather():
    return functools.partial(
        pl.kernel,
        mesh=plsc.VectorSubcoreMesh(
            core_axis_name="c", subcore_axis_name="s",
            num_cores=NC, num_subcores=NS),
        out_type=jax.ShapeDtypeStruct((B, PAD_DIM), jnp.float32),
        scratch_types=[
            pltpu.VMEM((NCHUNK, CHUNK), jnp.int32),
            pltpu.VMEM((NBUF, CHUNK, PAD_DIM), jnp.float32),
            pltpu.SemaphoreType.DMA((NBUF,)),
            pltpu.SemaphoreType.DMA((NBUF,)),
        ],
        compiler_params=pltpu.CompilerParams(use_tc_tiling_on_sc=True),
    )(_gather_body)


def kernel(input_ids, table):
    ids = input_ids.astype(jnp.int32).reshape(NW, NCHUNK, CHUNK)
    tpad = _make_tc_transpose()(table.T)
    out = _make_sc_gather()(tpad, ids)
    return out[:, :EMBED_DIM].reshape(BATCH, SEQ, EMBED_DIM)


# speedup vs baseline: 1.4819x; 1.0387x over previous
import functools

import jax
import jax.numpy as jnp
from jax import lax
from jax.experimental import pallas as pl
from jax.experimental.pallas import tpu as pltpu
from jax.experimental.pallas import tpu_sc as plsc

BATCH = 4096
SEQ = 50
EMBED_DIM = 64
PAD_DIM = 128
N_ROWS = 1000000

NC = 2
NS = 16
NW = NC * NS
B = BATCH * SEQ
B_PER_W = B // NW
CHUNK = 128
NCHUNK = B_PER_W // CHUNK
NBUF = 5
NOUTER = NCHUNK // NBUF

TCOLS = 16384
NTBLK = -(-N_ROWS // TCOLS)
PAD_ROWS = NTBLK * TCOLS


def _transpose_body(tt_ref, out_ref):
    x = tt_ref[...]
    out_ref[:, 0:EMBED_DIM] = x.T


@functools.cache
def _make_tc_transpose():
    return pl.pallas_call(
        _transpose_body,
        grid=(NTBLK,),
        in_specs=[pl.BlockSpec((EMBED_DIM, TCOLS), lambda i: (0, i))],
        out_specs=pl.BlockSpec((TCOLS, PAD_DIM), lambda i: (i, 0)),
        out_shape=jax.ShapeDtypeStruct((PAD_ROWS, PAD_DIM), jnp.float32),
    )


def _gather_body(table_hbm, idx_hbm, out_hbm, idx_v, rows_v, gsem, ssem):
    wid = lax.axis_index("s") * NC + lax.axis_index("c")
    base = wid * B_PER_W
    pltpu.sync_copy(idx_hbm.at[wid], idx_v)

    def outer(g, carry):
        for b in range(NBUF):
            @pl.when(g > 0)
            def _wait_store():
                pltpu.make_async_copy(
                    rows_v.at[b], out_hbm.at[pl.ds(base, CHUNK)],
                    ssem.at[b]).wait()
            pltpu.make_async_copy(
                table_hbm.at[idx_v.at[g * NBUF + b]], rows_v.at[b],
                gsem.at[b]).start()
        for b in range(NBUF):
            j = g * NBUF + b
            pltpu.make_async_copy(
                table_hbm.at[idx_v.at[j]], rows_v.at[b], gsem.at[b]).wait()
            pltpu.make_async_copy(
                rows_v.at[b], out_hbm.at[pl.ds(base + j * CHUNK, CHUNK)],
                ssem.at[b]).start()
        return carry

    lax.fori_loop(0, NOUTER, outer, 0)
    for b in range(NBUF):
        pltpu.make_async_copy(
            rows_v.at[b], out_hbm.at[pl.ds(base, CHUNK)], ssem.at[b]).wait()


@functools.cache
def _make_sc_gather():
    return functools.partial(
        pl.kernel,
        mesh=plsc.VectorSubcoreMesh(
            core_axis_name="c", subcore_axis_name="s",
            num_cores=NC, num_subcores=NS),
        out_type=jax.ShapeDtypeStruct((B, PAD_DIM), jnp.float32),
        scratch_types=[
            pltpu.VMEM((NCHUNK, CHUNK), jnp.int32),
            pltpu.VMEM((NBUF, CHUNK, PAD_DIM), jnp.float32),
            pltpu.SemaphoreType.DMA((NBUF,)),
            pltpu.SemaphoreType.DMA((NBUF,)),
        ],
        compiler_params=pltpu.CompilerParams(use_tc_tiling_on_sc=True),
    )(_gather_body)


def kernel(input_ids, table):
    ids = input_ids.astype(jnp.int32).reshape(NW, NCHUNK, CHUNK)
    tpad = _make_tc_transpose()(table.T)
    out = _make_sc_gather()(tpad, ids)
    return out[:, :EMBED_DIM].reshape(BATCH, SEQ, EMBED_DIM)


# speedup vs baseline: 1.5003x; 1.0124x over previous
import functools

import jax
import jax.numpy as jnp
from jax import lax
from jax.experimental import pallas as pl
from jax.experimental.pallas import tpu as pltpu
from jax.experimental.pallas import tpu_sc as plsc

BATCH = 4096
SEQ = 50
EMBED_DIM = 64
PAD_DIM = 128
N_ROWS = 1000000

NC = 2
NS = 16
NW = NC * NS
B = BATCH * SEQ
B_PER_W = B // NW
CHUNK = 128
NCHUNK = B_PER_W // CHUNK
NBUF = 5
NOUTER = NCHUNK // NBUF

TCOLS = 32768
NTBLK = -(-N_ROWS // TCOLS)
PAD_ROWS = NTBLK * TCOLS


def _transpose_body(tt_ref, out_ref):
    x = tt_ref[...]
    out_ref[:, 0:EMBED_DIM] = x.T


@functools.cache
def _make_tc_transpose():
    return pl.pallas_call(
        _transpose_body,
        grid=(NTBLK,),
        in_specs=[pl.BlockSpec((EMBED_DIM, TCOLS), lambda i: (0, i))],
        out_specs=pl.BlockSpec((TCOLS, PAD_DIM), lambda i: (i, 0)),
        out_shape=jax.ShapeDtypeStruct((PAD_ROWS, PAD_DIM), jnp.float32),
    )


def _gather_body(table_hbm, idx_hbm, out_hbm, idx_v, rows_v, gsem, ssem):
    wid = lax.axis_index("s") * NC + lax.axis_index("c")
    base = wid * B_PER_W
    pltpu.sync_copy(idx_hbm.at[wid], idx_v)

    def outer(g, carry):
        for b in range(NBUF):
            @pl.when(g > 0)
            def _wait_store():
                pltpu.make_async_copy(
                    rows_v.at[b], out_hbm.at[pl.ds(base, CHUNK)],
                    ssem.at[b]).wait()
            pltpu.make_async_copy(
                table_hbm.at[idx_v.at[g * NBUF + b]], rows_v.at[b],
                gsem.at[b]).start()
        for b in range(NBUF):
            j = g * NBUF + b
            pltpu.make_async_copy(
                table_hbm.at[idx_v.at[j]], rows_v.at[b], gsem.at[b]).wait()
            pltpu.make_async_copy(
                rows_v.at[b], out_hbm.at[pl.ds(base + j * CHUNK, CHUNK)],
                ssem.at[b]).start()
        return carry

    lax.fori_loop(0, NOUTER, outer, 0)
    for b in range(NBUF):
        pltpu.make_async_copy(
            rows_v.at[b], out_hbm.at[pl.ds(base, CHUNK)], ssem.at[b]).wait()


@functools.cache
def _make_sc_gather():
    return functools.partial(
        pl.kernel,
        mesh=plsc.VectorSubcoreMesh(
            core_axis_name="c", subcore_axis_name="s",
            num_cores=NC, num_subcores=NS),
        out_type=jax.ShapeDtypeStruct((B, PAD_DIM), jnp.float32),
        scratch_types=[
            pltpu.VMEM((NCHUNK, CHUNK), jnp.int32),
            pltpu.VMEM((NBUF, CHUNK, PAD_DIM), jnp.float32),
            pltpu.SemaphoreType.DMA((NBUF,)),
            pltpu.SemaphoreType.DMA((NBUF,)),
        ],
        compiler_params=pltpu.CompilerParams(use_tc_tiling_on_sc=True),
    )(_gather_body)


def kernel(input_ids, table):
    ids = input_ids.astype(jnp.int32).reshape(NW, NCHUNK, CHUNK)
    tpad = _make_tc_transpose()(table.T)
    out = _make_sc_gather()(tpad, ids)
    return out[:, :EMBED_DIM].reshape(BATCH, SEQ, EMBED_DIM)
